# Initial kernel scaffold; baseline (speedup 1.0000x reference)
import jax
import jax.numpy as jnp
from jax.experimental import pallas as pl


def kernel(h, edge_index, e, Uw, Ub, Vw, Vb, Aw, Ab, Bw, Bb, Cw, Cb, ln_h_w, ln_h_b, ln_e_w, ln_e_b):
    raise NotImplementedError("write your pallas kernel here")


# speedup vs baseline: 1.8520x; 1.8520x over previous
import functools

import jax
import jax.numpy as jnp
from jax import lax
from jax.experimental import pallas as pl
from jax.experimental.pallas import tpu as pltpu
from jax.experimental.pallas import tpu_sc as plsc

N = 10000
E = 160000
H = 256
HH = H // 2

NC = 2
NS = 16
EPT = E // NS
R = 80
NCHUNK = EPT // R
NPT = 624
NREM = N - NS * NPT


def _mm_node_body(h_ref, w_ref, b_ref, uh_ref, vlo_ref, vhi_ref, alo_ref,
                  ahi_ref, blo_ref, bhi_ref):
    acc = jnp.dot(h_ref[...], w_ref[...], preferred_element_type=jnp.float32)
    acc = acc + b_ref[...]
    uh_ref[...] = acc[:, 0:H]
    vlo_ref[...] = acc[:, H:H + HH]
    vhi_ref[...] = acc[:, H + HH:2 * H]
    alo_ref[...] = acc[:, 2 * H:2 * H + HH]
    ahi_ref[...] = acc[:, 2 * H + HH:3 * H]
    blo_ref[...] = acc[:, 3 * H:3 * H + HH]
    bhi_ref[...] = acc[:, 3 * H + HH:4 * H]


def _mm_node(h, w_all, b_all):
    bm = 1000
    grid = (N // bm,)
    f32 = jnp.float32
    outs = (
        jax.ShapeDtypeStruct((N, H), f32),
        jax.ShapeDtypeStruct((N, HH), f32), jax.ShapeDtypeStruct((N, HH), f32),
        jax.ShapeDtypeStruct((N, HH), f32), jax.ShapeDtypeStruct((N, HH), f32),
        jax.ShapeDtypeStruct((N, HH), f32), jax.ShapeDtypeStruct((N, HH), f32),
    )
    return pl.pallas_call(
        _mm_node_body,
        grid=grid,
        in_specs=[
            pl.BlockSpec((bm, H), lambda i: (i, 0)),
            pl.BlockSpec((H, 4 * H), lambda i: (0, 0)),
            pl.BlockSpec((1, 4 * H), lambda i: (0, 0)),
        ],
        out_specs=(
            pl.BlockSpec((bm, H), lambda i: (i, 0)),
            pl.BlockSpec((bm, HH), lambda i: (i, 0)),
            pl.BlockSpec((bm, HH), lambda i: (i, 0)),
            pl.BlockSpec((bm, HH), lambda i: (i, 0)),
            pl.BlockSpec((bm, HH), lambda i: (i, 0)),
            pl.BlockSpec((bm, HH), lambda i: (i, 0)),
            pl.BlockSpec((bm, HH), lambda i: (i, 0)),
        ),
        out_shape=outs,
    )(h, w_all, b_all)


def _mm_edge_body(e_ref, w_ref, b_ref, lo_ref, hi_ref):
    acc = jnp.dot(e_ref[...], w_ref[...], preferred_element_type=jnp.float32)
    acc = acc + b_ref[...]
    lo_ref[...] = acc[:, 0:HH]
    hi_ref[...] = acc[:, HH:H]


def _mm_edge(e, cw_t, cb):
    bm = 1000
    grid = (E // bm,)
    f32 = jnp.float32
    return pl.pallas_call(
        _mm_edge_body,
        grid=grid,
        in_specs=[
            pl.BlockSpec((bm, H), lambda i: (i, 0)),
            pl.BlockSpec((H, H), lambda i: (0, 0)),
            pl.BlockSpec((1, H), lambda i: (0, 0)),
        ],
        out_specs=(
            pl.BlockSpec((bm, HH), lambda i: (i, 0)),
            pl.BlockSpec((bm, HH), lambda i: (i, 0)),
        ),
        out_shape=(jax.ShapeDtypeStruct((E, HH), f32),
                   jax.ShapeDtypeStruct((E, HH), f32)),
    )(e, cw_t, cb)


def _sc_half(src_hbm, dst_hbm, v_t, a_t, b_t, ce_t, zeros_hbm, enew_t, agg_t,
             idx_s, idx_d, ah_v, bh_v, ce_v, vh_v, sem, agg_sh):
    sid = lax.axis_index("s")

    pltpu.sync_copy(zeros_hbm.at[pl.ds(sid * NPT, NPT)],
                    agg_sh.at[pl.ds(sid * NPT, NPT)])

    @pl.when(sid == NS - 1)
    def _():
        pltpu.sync_copy(zeros_hbm.at[pl.ds(NS * NPT, NREM)],
                        agg_sh.at[pl.ds(NS * NPT, NREM)])

    plsc.subcore_barrier()

    base0 = sid * EPT

    def chunk(i, carry):
        base = base0 + i * R
        pltpu.sync_copy(src_hbm.at[pl.ds(base, R)], idx_s)
        pltpu.sync_copy(dst_hbm.at[pl.ds(base, R)], idx_d)
        pltpu.sync_copy(ce_t.at[pl.ds(base, R)], ce_v)
        c1 = pltpu.async_copy(a_t.at[idx_s], ah_v, sem)
        c2 = pltpu.async_copy(b_t.at[idx_d], bh_v, sem)
        c3 = pltpu.async_copy(v_t.at[idx_s], vh_v, sem)
        c1.wait()
        c2.wait()
        c3.wait()

        def row(r, carry2):
            for j in range(HH // 16):
                sl = pl.ds(j * 16, 16)
                en = ah_v[r, sl] + bh_v[r, sl] + ce_v[r, sl]
                ce_v[r, sl] = en
                g = 1.0 / (1.0 + jnp.exp(-en))
                vh_v[r, sl] = g * vh_v[r, sl]
            return carry2

        lax.fori_loop(0, R, row, 0, unroll=False)

        pltpu.sync_copy(ce_v, enew_t.at[pl.ds(base, R)])
        pltpu.sync_copy(vh_v, agg_sh.at[idx_d], add=True)
        return carry

    lax.fori_loop(0, NCHUNK, chunk, 0, unroll=False)

    plsc.subcore_barrier()
    pltpu.sync_copy(agg_sh.at[pl.ds(sid * NPT, NPT)],
                    agg_t.at[pl.ds(sid * NPT, NPT)])

    @pl.when(sid == NS - 1)
    def _():
        pltpu.sync_copy(agg_sh.at[pl.ds(NS * NPT, NREM)],
                        agg_t.at[pl.ds(NS * NPT, NREM)])


def _sc_body(src_hbm, dst_hbm, vlo, vhi, alo, ahi, blo, bhi, celo, cehi,
             zeros_hbm, enew_lo, enew_hi, agg_lo, agg_hi,
             idx_s, idx_d, ah_v, bh_v, ce_v, vh_v, sem, agg_sh):
    cid = lax.axis_index("c")

    @pl.when(cid == 0)
    def _():
        _sc_half(src_hbm, dst_hbm, vlo, alo, blo, celo, zeros_hbm, enew_lo,
                 agg_lo, idx_s, idx_d, ah_v, bh_v, ce_v, vh_v, sem, agg_sh)

    @pl.when(cid == 1)
    def _():
        _sc_half(src_hbm, dst_hbm, vhi, ahi, bhi, cehi, zeros_hbm, enew_hi,
                 agg_hi, idx_s, idx_d, ah_v, bh_v, ce_v, vh_v, sem, agg_sh)


def _sc_edge_stage(src, dst, vlo, vhi, alo, ahi, blo, bhi, celo, cehi, zeros):
    f32 = jnp.float32
    i32 = jnp.int32
    mesh = plsc.VectorSubcoreMesh(core_axis_name="c", subcore_axis_name="s")
    out_type = (
        jax.ShapeDtypeStruct((E, HH), f32), jax.ShapeDtypeStruct((E, HH), f32),
        jax.ShapeDtypeStruct((N, HH), f32), jax.ShapeDtypeStruct((N, HH), f32),
    )
    scratch = [
        pltpu.VMEM((R,), i32), pltpu.VMEM((R,), i32),
        pltpu.VMEM((R, HH), f32), pltpu.VMEM((R, HH), f32),
        pltpu.VMEM((R, HH), f32), pltpu.VMEM((R, HH), f32),
        pltpu.SemaphoreType.DMA,
        pltpu.VMEM_SHARED((N, HH), f32),
    ]
    fn = pl.kernel(_sc_body, out_type=out_type, mesh=mesh,
                   scratch_types=scratch)
    return fn(src, dst, vlo, vhi, alo, ahi, blo, bhi, celo, cehi, zeros)


def _ln_relu_res(x, xn, w, b):
    m = jnp.mean(xn, axis=-1, keepdims=True)
    v = jnp.mean((xn - m) * (xn - m), axis=-1, keepdims=True)
    ln = (xn - m) / jnp.sqrt(v + 1e-5) * w + b
    return x + jnp.maximum(ln, 0.0)


def _h_epi_body(h_ref, uh_ref, alo_ref, ahi_ref, w_ref, b_ref, out_ref):
    hn = uh_ref[...] + jnp.concatenate([alo_ref[...], ahi_ref[...]], axis=-1)
    out_ref[...] = _ln_relu_res(h_ref[...], hn, w_ref[...], b_ref[...])


def _h_epilogue(h, uh, agg_lo, agg_hi, w, b):
    bm = 1000
    return pl.pallas_call(
        _h_epi_body,
        grid=(N // bm,),
        in_specs=[
            pl.BlockSpec((bm, H), lambda i: (i, 0)),
            pl.BlockSpec((bm, H), lambda i: (i, 0)),
            pl.BlockSpec((bm, HH), lambda i: (i, 0)),
            pl.BlockSpec((bm, HH), lambda i: (i, 0)),
            pl.BlockSpec((1, H), lambda i: (0, 0)),
            pl.BlockSpec((1, H), lambda i: (0, 0)),
        ],
        out_specs=pl.BlockSpec((bm, H), lambda i: (i, 0)),
        out_shape=jax.ShapeDtypeStruct((N, H), jnp.float32),
    )(h, uh, agg_lo, agg_hi, w, b)


def _e_epi_body(e_ref, nlo_ref, nhi_ref, w_ref, b_ref, out_ref):
    en = jnp.concatenate([nlo_ref[...], nhi_ref[...]], axis=-1)
    out_ref[...] = _ln_relu_res(e_ref[...], en, w_ref[...], b_ref[...])


def _e_epilogue(e, enew_lo, enew_hi, w, b):
    bm = 1000
    return pl.pallas_call(
        _e_epi_body,
        grid=(E // bm,),
        in_specs=[
            pl.BlockSpec((bm, H), lambda i: (i, 0)),
            pl.BlockSpec((bm, HH), lambda i: (i, 0)),
            pl.BlockSpec((bm, HH), lambda i: (i, 0)),
            pl.BlockSpec((1, H), lambda i: (0, 0)),
            pl.BlockSpec((1, H), lambda i: (0, 0)),
        ],
        out_specs=pl.BlockSpec((bm, H), lambda i: (i, 0)),
        out_shape=jax.ShapeDtypeStruct((E, H), jnp.float32),
    )(e, enew_lo, enew_hi, w, b)


def kernel(h, edge_index, e, Uw, Ub, Vw, Vb, Aw, Ab, Bw, Bb, Cw, Cb,
           ln_h_w, ln_h_b, ln_e_w, ln_e_b):
    f32 = jnp.float32
    dst = edge_index[0].astype(jnp.int32)
    src = edge_index[1].astype(jnp.int32)

    w_all = jnp.concatenate([Uw, Vw, Aw, Bw], axis=0).T
    b_all = jnp.concatenate([Ub, Vb, Ab, Bb]).reshape(1, 4 * H)

    uh, vlo, vhi, alo, ahi, blo, bhi = _mm_node(h, w_all, b_all)
    celo, cehi = _mm_edge(e, Cw.T, Cb.reshape(1, H))

    zeros = jnp.zeros((N, HH), dtype=f32)
    enew_lo, enew_hi, agg_lo, agg_hi = _sc_edge_stage(
        src, dst, vlo, vhi, alo, ahi, blo, bhi, celo, cehi, zeros)

    h_out = _h_epilogue(h, uh, agg_lo, agg_hi,
                        ln_h_w.reshape(1, H), ln_h_b.reshape(1, H))
    e_out = _e_epilogue(e, enew_lo, enew_hi,
                        ln_e_w.reshape(1, H), ln_e_b.reshape(1, H))
    return (h_out, e_out)
